# baseline (device time: 102015 ns/iter reference)
import jax
import jax.numpy as jnp
from jax import lax
from jax.experimental import pallas as pl
from jax.experimental.pallas import tpu as pltpu

N_DEV = 4


def kernel(x, w_mat, scale_x, scale_w):
    m_per, k = x.shape
    n_per = w_mat.shape[1]
    m_half = m_per // 2

    x8 = x.astype(jnp.float8_e4m3fn)
    w8 = w_mat.astype(jnp.float8_e5m2)

    def body(x_ref, w_ref, sx_ref, sw_ref, out_ref, xg_ref, send_sems, recv_sems):
        me = lax.axis_index("i")
        left = lax.rem(me + N_DEV - 1, N_DEV)
        right = lax.rem(me + 1, N_DEV)
        opp = lax.rem(me + 2, N_DEV)

        barrier_sem = pltpu.get_barrier_semaphore()
        for nbr in (left, right):
            pl.semaphore_signal(
                barrier_sem, inc=1,
                device_id=(nbr,), device_id_type=pl.DeviceIdType.MESH,
            )
        pl.semaphore_wait(barrier_sem, 2)

        def block(idx):
            return xg_ref.at[pl.ds(idx * m_per, m_per), :]

        def half(idx, h):
            return xg_ref.at[pl.ds(idx * m_per + h * m_half, m_half), :]

        def copy(src, dst, sem, dev):
            return pltpu.make_async_remote_copy(
                src_ref=src, dst_ref=dst,
                send_sem=send_sems.at[sem], recv_sem=recv_sems.at[sem],
                device_id=(dev,), device_id_type=pl.DeviceIdType.MESH,
            )

        x_lo = x_ref.at[pl.ds(0, m_half), :]
        x_hi = x_ref.at[pl.ds(m_half, m_half), :]

        s_r_lo = copy(x_lo, half(me, 0), 0, right)
        s_l_hi = copy(x_hi, half(me, 1), 3, left)
        s_r_hi = copy(x_hi, half(me, 1), 1, right)
        s_l_lo = copy(x_lo, half(me, 0), 4, left)
        s_r_lo.start()
        s_l_hi.start()
        s_r_hi.start()
        s_l_lo.start()

        scale = sx_ref[0] * sw_ref[0]
        w = w_ref[:, :]

        def mm(src_block, out_rows):
            acc = jnp.dot(
                src_block[:, :], w, preferred_element_type=jnp.float32,
            )
            out_ref[out_rows, :] = acc * scale

        mm(x_ref, pl.ds(me * m_per, m_per))

        r_left_lo = copy(x_lo, half(left, 0), 0, left)
        r_left_hi = copy(x_hi, half(left, 1), 1, left)
        r_right_hi = copy(x_hi, half(right, 1), 3, right)
        r_right_lo = copy(x_lo, half(right, 0), 4, right)
        r_opp_lo = copy(x_lo, half(opp, 0), 2, left)
        r_opp_hi = copy(x_hi, half(opp, 1), 5, right)

        r_left_lo.wait_recv()
        fwd_r = copy(half(left, 0), half(left, 0), 2, right)
        fwd_r.start()
        r_right_hi.wait_recv()
        fwd_l = copy(half(right, 1), half(right, 1), 5, left)
        fwd_l.start()

        r_left_hi.wait_recv()
        mm(block(left), pl.ds(left * m_per, m_per))
        r_right_lo.wait_recv()
        mm(block(right), pl.ds(right * m_per, m_per))

        r_opp_lo.wait_recv()
        r_opp_hi.wait_recv()
        mm(block(opp), pl.ds(opp * m_per, m_per))

        for s in (s_r_lo, s_l_hi, s_r_hi, s_l_lo, fwd_r, fwd_l):
            s.wait_send()

    return pl.pallas_call(
        body,
        out_shape=jax.ShapeDtypeStruct((N_DEV * m_per, n_per), jnp.float32),
        in_specs=[
            pl.BlockSpec(memory_space=pltpu.VMEM),
            pl.BlockSpec(memory_space=pltpu.VMEM),
            pl.BlockSpec(memory_space=pltpu.SMEM),
            pl.BlockSpec(memory_space=pltpu.SMEM),
        ],
        out_specs=pl.BlockSpec(memory_space=pltpu.VMEM),
        scratch_shapes=[
            pltpu.VMEM((N_DEV * m_per, k), jnp.float8_e4m3fn),
            pltpu.SemaphoreType.DMA((6,)),
            pltpu.SemaphoreType.DMA((6,)),
        ],
        compiler_params=pltpu.CompilerParams(
            collective_id=0, vmem_limit_bytes=96 * 1024 * 1024,
        ),
    )(x8, w8, scale_x, scale_w)
